# baseline (device time: 135342 ns/iter reference)
import jax
import jax.numpy as jnp
from jax import lax
from jax.experimental import pallas as pl
from jax.experimental.pallas import tpu as pltpu

N_EXP = 8
TOPK = 2


def _ag_xr_body(x_ref, r_ref, ox_ref, or_ref, send_sems, recv_sems):
    px = lax.axis_index("x")
    py = lax.axis_index("y")
    peer = (1 - px, py)

    ox_ref[pl.ds(px, 1)] = x_ref[...][None]
    or_ref[pl.ds(px, 1)] = r_ref[...][None]

    rdma_x = pltpu.make_async_remote_copy(
        src_ref=x_ref,
        dst_ref=ox_ref.at[px],
        send_sem=send_sems.at[0],
        recv_sem=recv_sems.at[0],
        device_id=peer,
        device_id_type=pl.DeviceIdType.MESH,
    )
    rdma_r = pltpu.make_async_remote_copy(
        src_ref=r_ref,
        dst_ref=or_ref.at[px],
        send_sem=send_sems.at[1],
        recv_sem=recv_sems.at[1],
        device_id=peer,
        device_id_type=pl.DeviceIdType.MESH,
    )
    rdma_x.start()
    rdma_r.start()
    rdma_x.wait()
    rdma_r.wait()


def _ag_xr(x_bf, r_shard):
    t_half, d = x_bf.shape
    t, e_loc = r_shard.shape
    return pl.pallas_call(
        _ag_xr_body,
        out_shape=(
            jax.ShapeDtypeStruct((2, t_half, d), jnp.bfloat16),
            jax.ShapeDtypeStruct((2, t, e_loc), jnp.float32),
        ),
        in_specs=[
            pl.BlockSpec(memory_space=pltpu.VMEM),
            pl.BlockSpec(memory_space=pltpu.VMEM),
        ],
        out_specs=(
            pl.BlockSpec(memory_space=pltpu.VMEM),
            pl.BlockSpec(memory_space=pltpu.VMEM),
        ),
        scratch_shapes=[
            pltpu.SemaphoreType.DMA((2,)),
            pltpu.SemaphoreType.DMA((2,)),
        ],
    )(x_bf, r_shard)


def _ag_w_body(w_ref, ow_ref, send_sem, recv_sem):
    px = lax.axis_index("x")
    py = lax.axis_index("y")
    ow_ref[pl.ds(px, 1)] = w_ref[...][None]
    rdma = pltpu.make_async_remote_copy(
        src_ref=w_ref,
        dst_ref=ow_ref.at[px],
        send_sem=send_sem,
        recv_sem=recv_sem,
        device_id=(1 - px, py),
        device_id_type=pl.DeviceIdType.MESH,
    )
    rdma.start()
    rdma.wait()


def _ag_w(w_own):
    t_half, ne = w_own.shape
    return pl.pallas_call(
        _ag_w_body,
        out_shape=jax.ShapeDtypeStruct((2, t_half, ne), jnp.float32),
        in_specs=[pl.BlockSpec(memory_space=pltpu.VMEM)],
        out_specs=pl.BlockSpec(memory_space=pltpu.VMEM),
        scratch_shapes=[
            pltpu.SemaphoreType.DMA(()),
            pltpu.SemaphoreType.DMA(()),
        ],
    )(w_own)


def _moe_body(x_ref, w1_ref, w2_ref, wg_ref, out_ref):
    e = pl.program_id(0)
    h = jnp.dot(
        x_ref[...], w1_ref[0], preferred_element_type=jnp.float32
    )
    h = jnp.maximum(h, 0.0).astype(jnp.bfloat16)
    contrib = jnp.dot(
        h, w2_ref[0], preferred_element_type=jnp.float32
    )
    wg = wg_ref[...]
    col_ids = lax.broadcasted_iota(jnp.int32, wg.shape, 1)
    col = jnp.sum(jnp.where(col_ids == e, wg, 0.0), axis=1, keepdims=True)
    contrib = contrib * col

    @pl.when(e == 0)
    def _():
        out_ref[...] = contrib

    @pl.when(e != 0)
    def _():
        out_ref[...] += contrib


def _moe_compute(x_full, W1, W2, w_loc):
    t, d = x_full.shape
    e_loc, _, f = W1.shape
    return pl.pallas_call(
        _moe_body,
        grid=(e_loc,),
        in_specs=[
            pl.BlockSpec((t, d), lambda e: (0, 0)),
            pl.BlockSpec((1, d, f), lambda e: (e, 0, 0)),
            pl.BlockSpec((1, f, d), lambda e: (e, 0, 0)),
            pl.BlockSpec((t, e_loc), lambda e: (0, 0)),
        ],
        out_specs=pl.BlockSpec((t, d), lambda e: (0, 0)),
        out_shape=jax.ShapeDtypeStruct((t, d), jnp.float32),
        compiler_params=pltpu.CompilerParams(
            dimension_semantics=("arbitrary",),
            vmem_limit_bytes=100 * 1024 * 1024,
        ),
    )(x_full, W1, W2, w_loc)


def _combine_body(p_ref, out_ref, send_buf, recv_buf, send_sem, recv_sem):
    px = lax.axis_index("x")
    py = lax.axis_index("y")
    peer_x = 1 - px
    t_half = out_ref.shape[0]

    send_buf[...] = p_ref[pl.ds(peer_x * t_half, t_half), :].astype(
        jnp.bfloat16
    )
    rdma = pltpu.make_async_remote_copy(
        src_ref=send_buf,
        dst_ref=recv_buf,
        send_sem=send_sem,
        recv_sem=recv_sem,
        device_id=(peer_x, py),
        device_id_type=pl.DeviceIdType.MESH,
    )
    rdma.start()
    rdma.wait()
    out_ref[...] = p_ref[pl.ds(px * t_half, t_half), :] + recv_buf[
        ...
    ].astype(jnp.float32)


def _combine(partial, t_half):
    t, d = partial.shape
    return pl.pallas_call(
        _combine_body,
        out_shape=jax.ShapeDtypeStruct((t_half, d), jnp.float32),
        in_specs=[pl.BlockSpec(memory_space=pltpu.VMEM)],
        out_specs=pl.BlockSpec(memory_space=pltpu.VMEM),
        scratch_shapes=[
            pltpu.VMEM((t_half, d), jnp.bfloat16),
            pltpu.VMEM((t_half, d), jnp.bfloat16),
            pltpu.SemaphoreType.DMA(()),
            pltpu.SemaphoreType.DMA(()),
        ],
    )(partial)


def kernel(x, router, W1, W2):
    t_half, d = x.shape
    e_loc = W1.shape[0]
    px = lax.axis_index("x")

    xg, rg = _ag_xr(x.astype(jnp.bfloat16), router)
    x_full = xg.reshape(2 * t_half, d)
    router_full = jnp.concatenate([rg[0], rg[1]], axis=1)

    gates = jnp.dot(x, router_full, precision=lax.Precision.HIGHEST)
    top_v, top_i = lax.top_k(gates, TOPK)
    wts = jax.nn.softmax(top_v, axis=-1)
    one_hot = top_i[:, :, None] == jnp.arange(N_EXP)[None, None, :]
    w_own = jnp.sum(
        jnp.where(one_hot, wts[:, :, None], 0.0), axis=1
    )

    wg = _ag_w(w_own)
    w_full = wg.reshape(2 * t_half, N_EXP)
    w_loc = lax.dynamic_slice_in_dim(w_full, px * e_loc, e_loc, axis=1)

    partial = _moe_compute(
        x_full, W1.astype(jnp.bfloat16), W2.astype(jnp.bfloat16), w_loc
    )

    return _combine(partial, t_half)


# device time: 86920 ns/iter; 1.5571x vs baseline; 1.5571x over previous
import jax
import jax.numpy as jnp
from jax import lax
from jax.experimental import pallas as pl
from jax.experimental.pallas import tpu as pltpu

N_EXP = 8
E_LOC = 4


def _wcol(w, c):
    ids = lax.broadcasted_iota(jnp.int32, w.shape, 1)
    return jnp.sum(jnp.where(ids == c, w, 0.0), axis=1, keepdims=True)


def _body(
    x_ref, r_ref, w1_ref, w2_ref, out_ref,
    xloc, xg, rg, wsend, wrecv,
    p_peer, psend, precv,
    send_sems, recv_sems,
):
    s = pl.program_id(0)
    px = lax.axis_index("x")
    py = lax.axis_index("y")
    peer = (1 - px, py)

    def mk(src, dst, i):
        return pltpu.make_async_remote_copy(
            src_ref=src,
            dst_ref=dst,
            send_sem=send_sems.at[i],
            recv_sem=recv_sems.at[i],
            device_id=peer,
            device_id_type=pl.DeviceIdType.MESH,
        )

    rdma_x = mk(xloc, xg, 0)
    rdma_r = mk(r_ref, rg, 1)
    rdma_w = mk(wsend, wrecv, 2)
    rdma_p = mk(psend, precv, 3)

    w1_bf = w1_ref[0].astype(jnp.bfloat16)
    w2_bf = w2_ref[0].astype(jnp.bfloat16)

    def expert(x_bf, wcol):
        h = jnp.dot(x_bf, w1_bf, preferred_element_type=jnp.float32)
        h = jnp.maximum(h, 0.0).astype(jnp.bfloat16)
        return jnp.dot(h, w2_bf, preferred_element_type=jnp.float32) * wcol

    def own(e):
        return expert(xloc[...], _wcol(wsend[...], E_LOC * px + e))

    def peer_half(e):
        return expert(xg[...], _wcol(wrecv[...], E_LOC * px + e))

    @pl.when(s == 0)
    def _s0():
        barrier_sem = pltpu.get_barrier_semaphore()
        pl.semaphore_signal(
            barrier_sem, inc=1, device_id=peer,
            device_id_type=pl.DeviceIdType.MESH,
        )
        pl.semaphore_wait(barrier_sem, 1)

        xloc[...] = x_ref[...].astype(jnp.bfloat16)
        rdma_x.start()
        rdma_r.start()

        rdma_r.wait()
        g_mine = jnp.dot(x_ref[...], r_ref[...],
                         precision=lax.Precision.HIGHEST,
                         preferred_element_type=jnp.float32)
        g_peer = jnp.dot(x_ref[...], rg[...],
                         precision=lax.Precision.HIGHEST,
                         preferred_element_type=jnp.float32)
        g = jnp.where(
            px == 0,
            jnp.concatenate([g_mine, g_peer], axis=1),
            jnp.concatenate([g_peer, g_mine], axis=1),
        )
        m1 = jnp.max(g, axis=1, keepdims=True)
        oh1 = g == m1
        g2 = jnp.where(oh1, -jnp.inf, g)
        m2 = jnp.max(g2, axis=1, keepdims=True)
        oh2 = g2 == m2
        e2 = jnp.exp(m2 - m1)
        wa = 1.0 / (1.0 + e2)
        wb = e2 / (1.0 + e2)
        wsend[...] = jnp.where(oh1, wa, 0.0) + jnp.where(oh2, wb, 0.0)
        rdma_w.start()

        out_ref[...] = own(0)

        rdma_x.wait()
        rdma_w.wait()
        p_peer[...] = peer_half(0)

    @pl.when(s == 1)
    def _s1():
        out_ref[...] += own(1)
        p_peer[...] += peer_half(1)

    @pl.when(s == 2)
    def _s2():
        p_peer[...] += peer_half(2)

    @pl.when(s == 3)
    def _s3():
        p_peer[...] += peer_half(3)
        psend[...] = p_peer[...].astype(jnp.bfloat16)
        rdma_p.start()

    @pl.when(s == 4)
    def _s4():
        out_ref[...] += own(2)

    @pl.when(s == 5)
    def _s5():
        out_ref[...] += own(3)
        rdma_p.wait()
        out_ref[...] += precv[...].astype(jnp.float32)


def kernel(x, router, W1, W2):
    t_half, d = x.shape
    e_loc, _, f = W1.shape

    def w_idx(s):
        return (s - 2 * (s // 4), 0, 0)

    return pl.pallas_call(
        _body,
        grid=(6,),
        in_specs=[
            pl.BlockSpec((t_half, d), lambda s: (0, 0)),
            pl.BlockSpec((2 * t_half, e_loc), lambda s: (0, 0)),
            pl.BlockSpec((1, d, f), w_idx),
            pl.BlockSpec((1, f, d), w_idx),
        ],
        out_specs=pl.BlockSpec((t_half, d), lambda s: (0, 0)),
        out_shape=jax.ShapeDtypeStruct((t_half, d), jnp.float32),
        scratch_shapes=[
            pltpu.VMEM((t_half, d), jnp.bfloat16),
            pltpu.VMEM((t_half, d), jnp.bfloat16),
            pltpu.VMEM((2 * t_half, e_loc), jnp.float32),
            pltpu.VMEM((t_half, N_EXP), jnp.float32),
            pltpu.VMEM((t_half, N_EXP), jnp.float32),
            pltpu.VMEM((t_half, d), jnp.float32),
            pltpu.VMEM((t_half, d), jnp.bfloat16),
            pltpu.VMEM((t_half, d), jnp.bfloat16),
            pltpu.SemaphoreType.DMA((4,)),
            pltpu.SemaphoreType.DMA((4,)),
        ],
        compiler_params=pltpu.CompilerParams(
            dimension_semantics=("arbitrary",),
            collective_id=0,
            vmem_limit_bytes=64 * 1024 * 1024,
        ),
    )(x, router, W1, W2)
